# Initial kernel scaffold; baseline (speedup 1.0000x reference)
import jax
import jax.numpy as jnp
from jax.experimental import pallas as pl


def kernel(rho, mask_x, mask_y, new_x, new_y):
    raise NotImplementedError("write your pallas kernel here")


# speedup vs baseline: 4.0677x; 4.0677x over previous
import jax
import jax.numpy as jnp
from jax import lax
from jax.experimental import pallas as pl
from jax.experimental.pallas import tpu as pltpu
from jax.experimental.pallas import tpu_sc as plsc

_CH = 16
_HALF = 128
_NCHUNK = _HALF // _CH


def _qpool_body(rp_hbm, out_hbm, rbuf, cbuf, dbuf, obuf, sem0, sem1, sem2):
    cid = lax.axis_index("c")
    sid = lax.axis_index("s")
    lanes = lax.iota(jnp.int32, 16)

    def chunk_body(k, carry):
        i0 = 8 * cid + k
        x0 = 16 * i0
        rp0 = sid * 512 + 32 * i0
        cp0 = pltpu.async_copy(rp_hbm.at[pl.ds(rp0, 16)], rbuf, sem0)
        cp1 = pltpu.async_copy(
            rp_hbm.at[pl.ds(rp0 + 16, 16), pl.ds(64 * i0 + 32, 32)],
            cbuf, sem1)
        cp2 = pltpu.async_copy(
            rp_hbm.at[pl.ds(rp0 + 16, 16), pl.ds(1024 + 64 * i0 + 32, 32)],
            dbuf, sem2)
        cp0.wait()
        cp1.wait()
        cp2.wait()

        def row_body(t, carry2):
            tf = jnp.full((16,), t, jnp.int32)

            def lp_body(lp, c3):
                av = plsc.load_gather(rbuf, [tf, 64 * lp + 2 * lanes])
                plsc.store_scatter(obuf, [tf, 16 * lp + lanes], av)
                return c3
            lax.fori_loop(0, 16, lp_body, 0)

            bv = plsc.load_gather(rbuf, [tf, 1024 + 64 * lanes + 2 * t + 1])
            plsc.addupdate_scatter(obuf, [tf, 16 * lanes + t], bv)

            cv = plsc.load_gather(cbuf, [tf, 2 * lanes])
            dv = plsc.load_gather(dbuf, [tf,
                                         jnp.full((16,), 2 * t + 1,
                                                  jnp.int32)])
            cd = cv + jnp.where(lanes == t, dv, jnp.float32(0))
            plsc.addupdate_scatter(obuf, [tf, 16 * i0 + lanes], cd)
            return carry2
        lax.fori_loop(0, _CH, row_body, 0)

        orow = sid * 256 + x0
        pltpu.sync_copy(obuf, out_hbm.at[pl.ds(orow, _CH)])
        return carry
    lax.fori_loop(0, _NCHUNK, chunk_body, 0)


def kernel(rho, mask_x, mask_y, new_x, new_y):
    b = rho.shape[0]
    rp = rho.reshape(b * 512, 2048)

    f = pl.kernel(
        _qpool_body,
        out_type=jax.ShapeDtypeStruct((b * 256, 256), jnp.float32),
        mesh=plsc.VectorSubcoreMesh(core_axis_name="c", subcore_axis_name="s"),
        scratch_types=[
            pltpu.VMEM((_CH, 2048), jnp.float32),
            pltpu.VMEM((_CH, 32), jnp.float32),
            pltpu.VMEM((_CH, 32), jnp.float32),
            pltpu.VMEM((_CH, 256), jnp.float32),
            pltpu.SemaphoreType.DMA,
            pltpu.SemaphoreType.DMA,
            pltpu.SemaphoreType.DMA,
        ],
        compiler_params=pltpu.CompilerParams(use_tc_tiling_on_sc=False,
                                             needs_layout_passes=False),
    )
    out = f(rp)
    return out.reshape(b, 256, 256)
